# baseline (speedup 1.0000x reference)
import functools
import jax
import jax.numpy as jnp
from jax import lax
from jax.experimental import pallas as pl
from jax.experimental.pallas import tpu as pltpu
from jax.experimental.pallas import tpu_sc as plsc

D = 64
SCALE = 8.0

NUM_CORES = 2
NUM_SUBCORES = 16
NW = NUM_CORES * NUM_SUBCORES

BATCH = 4096
SEQ = 200
BW = BATCH // NW
VOCAB = 1000000
VMAIN = 999936
VOCAB2 = VOCAB // 2

C1 = 512
NCH = VMAIN // C1
P1_ITER = NCH // NW + 1


def _format_fn():
    mesh = plsc.VectorSubcoreMesh(core_axis_name="c", subcore_axis_name="s")

    @functools.partial(
        pl.kernel,
        out_type=jax.ShapeDtypeStruct((VOCAB2, 2 * D), jnp.float32),
        mesh=mesh,
        scratch_types=[
            pltpu.VMEM((D, C1), jnp.float32),
            pltpu.VMEM((D, C1), jnp.float32),
            pltpu.VMEM((C1 // 2, 2 * D), jnp.float32),
            pltpu.VMEM((D, D), jnp.float32),
            pltpu.VMEM((D // 2, 2 * D), jnp.float32),
            pltpu.SemaphoreType.DMA,
            pltpu.SemaphoreType.DMA,
        ],
        compiler_params=pltpu.CompilerParams(
            use_tc_tiling_on_sc=True, needs_layout_passes=False),
    )
    def fmt_kernel(tT_hbm, t2_hbm, blk0, blk1, trans, tailblk, tailtr,
                   sem0, sem1):
        wid = lax.axis_index("s") * NUM_CORES + lax.axis_index("c")
        blks = (blk0, blk1)
        sems = (sem0, sem1)

        def fire(c, slot):
            pltpu.async_copy(
                tT_hbm.at[:, pl.ds(c * C1, C1)], blks[slot], sems[slot])

        def wait(slot):
            pltpu.make_async_copy(
                tT_hbm.at[:, pl.ds(0, C1)], blks[slot], sems[slot]).wait()

        def transform_write(c, slot):
            def pair(p, _):
                for k in range(D // 16):
                    rids = lax.iota(jnp.int32, 16) + (k * 16)
                    c0 = jnp.full((16,), 0, jnp.int32) + 2 * p
                    v0 = plsc.load_gather(blks[slot], [rids, c0])
                    trans[p, pl.ds(k * 16, 16)] = v0 * SCALE
                    v1 = plsc.load_gather(blks[slot], [rids, c0 + 1])
                    trans[p, pl.ds(D + k * 16, 16)] = v1 * SCALE
                return 0

            lax.fori_loop(0, C1 // 2, pair, 0)
            pltpu.sync_copy(trans, t2_hbm.at[pl.ds(c * (C1 // 2), C1 // 2)])

        @pl.when(wid < NCH)
        def _():
            fire(wid, 0)

        def pair_body(g, _):
            for b in range(2):
                j = 2 * g + b
                c = wid + NW * j
                nxt = c + NW

                @pl.when(nxt < NCH)
                def _():
                    fire(nxt, 1 - b)

                @pl.when(c < NCH)
                def _():
                    wait(b)
                    transform_write(c, b)
            return 0

        lax.fori_loop(0, P1_ITER // 2, pair_body, 0)

        @pl.when(wid == 1)
        def _():
            pltpu.sync_copy(tT_hbm.at[:, pl.ds(VMAIN, D)], tailblk)

            def tpair(p, _):
                for k in range(D // 16):
                    rids = lax.iota(jnp.int32, 16) + (k * 16)
                    c0 = jnp.full((16,), 0, jnp.int32) + 2 * p
                    v0 = plsc.load_gather(tailblk, [rids, c0])
                    tailtr[p, pl.ds(k * 16, 16)] = v0 * SCALE
                    v1 = plsc.load_gather(tailblk, [rids, c0 + 1])
                    tailtr[p, pl.ds(D + k * 16, 16)] = v1 * SCALE
                return 0

            lax.fori_loop(0, D // 2, tpair, 0)
            pltpu.sync_copy(tailtr, t2_hbm.at[pl.ds(VMAIN // 2, D // 2)])

    return fmt_kernel


def _gather_fn():
    mesh = plsc.VectorSubcoreMesh(core_axis_name="c", subcore_axis_name="s")

    @functools.partial(
        pl.kernel,
        out_type=jax.ShapeDtypeStruct((SEQ, D, BATCH), jnp.float32),
        mesh=mesh,
        scratch_types=[
            pltpu.VMEM((BW,), jnp.int32),
            pltpu.VMEM((BW,), jnp.int32),
            pltpu.VMEM((BW,), jnp.int32),
            pltpu.VMEM((BW,), jnp.int32),
            pltpu.VMEM((BW, 2 * D), jnp.float32),
            pltpu.VMEM((BW, 2 * D), jnp.float32),
            pltpu.VMEM((D, BW), jnp.float32),
            pltpu.SemaphoreType.DMA,
            pltpu.SemaphoreType.DMA,
        ],
        compiler_params=pltpu.CompilerParams(
            use_tc_tiling_on_sc=True, needs_layout_passes=False),
    )
    def enc_kernel(xt_hbm, table2_hbm, out_hbm,
                   idx0, idx1, idx2a, idx2b, rows0, rows1, trans, sem0, sem1):
        wid = lax.axis_index("s") * NUM_CORES + lax.axis_index("c")
        b0 = wid * BW
        idxs = (idx0, idx1)
        idx2s = (idx2a, idx2b)
        rows = (rows0, rows1)
        sems = (sem0, sem1)

        def fire(s, slot):
            pltpu.sync_copy(xt_hbm.at[pl.ds(s * BATCH + b0, BW)], idxs[slot])
            for k in range(BW // 16):
                sl = pl.ds(k * 16, 16)
                idx2s[slot][sl] = lax.shift_right_logical(idxs[slot][sl], 1)
            pltpu.async_copy(table2_hbm.at[idx2s[slot]], rows[slot], sems[slot])

        def wait(slot):
            pltpu.make_async_copy(
                table2_hbm.at[idx2s[slot]], rows[slot], sems[slot]).wait()

        def transform_write(s, slot):
            for k in range(BW // 16):
                rids = lax.iota(jnp.int32, 16) + (k * 16)
                half = lax.shift_left(
                    jnp.bitwise_and(idxs[slot][pl.ds(k * 16, 16)], 1), 6)
                for d in range(D):
                    v = plsc.load_gather(rows[slot], [rids, half + d])
                    trans[d, pl.ds(k * 16, 16)] = v
            pltpu.sync_copy(trans, out_hbm.at[s, :, pl.ds(b0, BW)])

        fire(0, 0)

        def pair_body(g, _):
            for b in range(2):
                s = 2 * g + b

                @pl.when(s + 1 < SEQ)
                def _():
                    fire(s + 1, 1 - b)

                wait(b)
                transform_write(s, b)
            return 0

        lax.fori_loop(0, SEQ // 2, pair_body, 0)

    return enc_kernel


_FORMAT = _format_fn()
_ENCODER = _gather_fn()


def kernel(x, table):
    xt = jnp.transpose(x, (1, 0)).reshape(-1).astype(jnp.int32)
    table2 = _FORMAT(jnp.transpose(table, (1, 0)))
    out_t = _ENCODER(xt, table2)
    return jnp.transpose(out_t, (0, 2, 1))


# speedup vs baseline: 1.1052x; 1.1052x over previous
import functools
import jax
import jax.numpy as jnp
from jax import lax
from jax.experimental import pallas as pl
from jax.experimental.pallas import tpu as pltpu
from jax.experimental.pallas import tpu_sc as plsc

D = 64
SCALE = 8.0

NUM_CORES = 2
NUM_SUBCORES = 16
NW = NUM_CORES * NUM_SUBCORES

BATCH = 4096
SEQ = 200
BW = BATCH // NW
VOCAB = 1000000
VMAIN = 999936
VOCAB2 = VOCAB // 2

C1 = 512
NCH = VMAIN // C1
P1_ITER = NCH // NW + 1


def _format_fn():
    mesh = plsc.VectorSubcoreMesh(core_axis_name="c", subcore_axis_name="s")

    @functools.partial(
        pl.kernel,
        out_type=jax.ShapeDtypeStruct((VOCAB2, 2 * D), jnp.float32),
        mesh=mesh,
        scratch_types=[
            pltpu.VMEM((D, C1 + 1), jnp.float32),
            pltpu.VMEM((D, C1 + 1), jnp.float32),
            pltpu.VMEM((C1 // 2, 2 * D), jnp.float32),
            pltpu.VMEM((D, D), jnp.float32),
            pltpu.VMEM((D // 2, 2 * D), jnp.float32),
            pltpu.SemaphoreType.DMA,
            pltpu.SemaphoreType.DMA,
        ],
        compiler_params=pltpu.CompilerParams(
            use_tc_tiling_on_sc=True, needs_layout_passes=False),
    )
    def fmt_kernel(tT_hbm, t2_hbm, blk0, blk1, trans, tailblk, tailtr,
                   sem0, sem1):
        wid = lax.axis_index("s") * NUM_CORES + lax.axis_index("c")
        blks = (blk0, blk1)
        sems = (sem0, sem1)

        def fire(c, slot):
            pltpu.async_copy(
                tT_hbm.at[:, pl.ds(c * C1, C1)],
                blks[slot].at[:, pl.ds(0, C1)], sems[slot])

        def wait(slot):
            pltpu.make_async_copy(
                tT_hbm.at[:, pl.ds(0, C1)],
                blks[slot].at[:, pl.ds(0, C1)], sems[slot]).wait()

        def transform_write(c, slot):
            def pair(p, _):
                for k in range(D // 16):
                    rids = lax.iota(jnp.int32, 16) + (k * 16)
                    c0 = jnp.full((16,), 0, jnp.int32) + 2 * p
                    v0 = plsc.load_gather(blks[slot], [rids, c0])
                    trans[p, pl.ds(k * 16, 16)] = v0 * SCALE
                    v1 = plsc.load_gather(blks[slot], [rids, c0 + 1])
                    trans[p, pl.ds(D + k * 16, 16)] = v1 * SCALE
                return 0

            lax.fori_loop(0, C1 // 2, pair, 0)
            pltpu.sync_copy(trans, t2_hbm.at[pl.ds(c * (C1 // 2), C1 // 2)])

        @pl.when(wid < NCH)
        def _():
            fire(wid, 0)

        def pair_body(g, _):
            for b in range(2):
                j = 2 * g + b
                c = wid + NW * j
                nxt = c + NW

                @pl.when(nxt < NCH)
                def _():
                    fire(nxt, 1 - b)

                @pl.when(c < NCH)
                def _():
                    wait(b)
                    transform_write(c, b)
            return 0

        lax.fori_loop(0, P1_ITER // 2, pair_body, 0)

        @pl.when(wid == 1)
        def _():
            pltpu.sync_copy(tT_hbm.at[:, pl.ds(VMAIN, D)], tailblk)

            def tpair(p, _):
                for k in range(D // 16):
                    rids = lax.iota(jnp.int32, 16) + (k * 16)
                    c0 = jnp.full((16,), 0, jnp.int32) + 2 * p
                    v0 = plsc.load_gather(tailblk, [rids, c0])
                    tailtr[p, pl.ds(k * 16, 16)] = v0 * SCALE
                    v1 = plsc.load_gather(tailblk, [rids, c0 + 1])
                    tailtr[p, pl.ds(D + k * 16, 16)] = v1 * SCALE
                return 0

            lax.fori_loop(0, D // 2, tpair, 0)
            pltpu.sync_copy(tailtr, t2_hbm.at[pl.ds(VMAIN // 2, D // 2)])

    return fmt_kernel


def _gather_fn():
    mesh = plsc.VectorSubcoreMesh(core_axis_name="c", subcore_axis_name="s")

    @functools.partial(
        pl.kernel,
        out_type=jax.ShapeDtypeStruct((SEQ, D, BATCH), jnp.float32),
        mesh=mesh,
        scratch_types=[
            pltpu.VMEM((BW,), jnp.int32),
            pltpu.VMEM((BW,), jnp.int32),
            pltpu.VMEM((BW,), jnp.int32),
            pltpu.VMEM((BW,), jnp.int32),
            pltpu.VMEM((BW, 2 * D), jnp.float32),
            pltpu.VMEM((BW, 2 * D), jnp.float32),
            pltpu.VMEM((D, BW + 1), jnp.float32),
            pltpu.SemaphoreType.DMA,
            pltpu.SemaphoreType.DMA,
        ],
        compiler_params=pltpu.CompilerParams(
            use_tc_tiling_on_sc=True, needs_layout_passes=False),
    )
    def enc_kernel(xt_hbm, table2_hbm, out_hbm,
                   idx0, idx1, idx2a, idx2b, rows0, rows1, trans, sem0, sem1):
        wid = lax.axis_index("s") * NUM_CORES + lax.axis_index("c")
        b0 = wid * BW
        idxs = (idx0, idx1)
        idx2s = (idx2a, idx2b)
        rows = (rows0, rows1)
        sems = (sem0, sem1)

        def fire(s, slot):
            pltpu.sync_copy(xt_hbm.at[pl.ds(s * BATCH + b0, BW)], idxs[slot])
            for k in range(BW // 16):
                sl = pl.ds(k * 16, 16)
                idx2s[slot][sl] = lax.shift_right_logical(idxs[slot][sl], 1)
            pltpu.async_copy(table2_hbm.at[idx2s[slot]], rows[slot], sems[slot])

        def wait(slot):
            pltpu.make_async_copy(
                table2_hbm.at[idx2s[slot]], rows[slot], sems[slot]).wait()

        def transform_write(s, slot):
            def cgroup(g, _):
                idx16 = jnp.bitwise_and(idxs[slot][pl.ds(g * 16, 16)], 1) * D
                for q in range(16):
                    c = g * 16 + q
                    half = idx16[q]
                    for k in range(D // 16):
                        v = rows[slot][c, pl.ds(half + k * 16, 16)]
                        rids = lax.iota(jnp.int32, 16) + (k * 16)
                        cids = jnp.full((16,), 0, jnp.int32) + c
                        plsc.store_scatter(trans, [rids, cids], v)
                return 0

            lax.fori_loop(0, BW // 16, cgroup, 0)
            pltpu.sync_copy(
                trans.at[:, pl.ds(0, BW)], out_hbm.at[s, :, pl.ds(b0, BW)])

        fire(0, 0)

        def pair_body(g, _):
            for b in range(2):
                s = 2 * g + b

                @pl.when(s + 1 < SEQ)
                def _():
                    fire(s + 1, 1 - b)

                wait(b)
                transform_write(s, b)
            return 0

        lax.fori_loop(0, SEQ // 2, pair_body, 0)

    return enc_kernel


_FORMAT = _format_fn()
_ENCODER = _gather_fn()


def kernel(x, table):
    xt = jnp.transpose(x, (1, 0)).reshape(-1).astype(jnp.int32)
    table2 = _FORMAT(jnp.transpose(table, (1, 0)))
    out_t = _ENCODER(xt, table2)
    return jnp.transpose(out_t, (0, 2, 1))


# speedup vs baseline: 2.4943x; 2.2568x over previous
import functools
import jax
import jax.numpy as jnp
from jax import lax
from jax.experimental import pallas as pl
from jax.experimental.pallas import tpu as pltpu
from jax.experimental.pallas import tpu_sc as plsc

D = 64
SCALE = 8.0

NUM_CORES = 2
NUM_SUBCORES = 16
NW = NUM_CORES * NUM_SUBCORES

BATCH = 4096
SEQ = 200
ROWS = BATCH * SEQ
ROWS_PER_W = ROWS // NW
CHUNK = 800
NCHUNK = ROWS_PER_W // CHUNK


def _gather_fn():
    mesh = plsc.VectorSubcoreMesh(core_axis_name="c", subcore_axis_name="s")

    @functools.partial(
        pl.kernel,
        out_type=jax.ShapeDtypeStruct((ROWS, D), jnp.float32),
        mesh=mesh,
        scratch_types=[
            pltpu.VMEM((CHUNK,), jnp.int32),
            pltpu.VMEM((CHUNK,), jnp.int32),
            pltpu.VMEM((CHUNK, D), jnp.float32),
            pltpu.VMEM((CHUNK, D), jnp.float32),
            pltpu.SemaphoreType.DMA,
            pltpu.SemaphoreType.DMA,
        ],
        compiler_params=pltpu.CompilerParams(
            use_tc_tiling_on_sc=False, needs_layout_passes=False),
    )
    def gather_kernel(idx_hbm, table_hbm, out_hbm,
                      idx0, idx1, rows0, rows1, sem0, sem1):
        wid = lax.axis_index("s") * NUM_CORES + lax.axis_index("c")
        base = wid * ROWS_PER_W
        idxs = (idx0, idx1)
        rows = (rows0, rows1)
        sems = (sem0, sem1)

        def fire(i, slot):
            pltpu.sync_copy(idx_hbm.at[pl.ds(base + i * CHUNK, CHUNK)],
                            idxs[slot])
            pltpu.async_copy(table_hbm.at[idxs[slot]], rows[slot], sems[slot])

        def wait(slot):
            pltpu.make_async_copy(
                table_hbm.at[idxs[slot]], rows[slot], sems[slot]).wait()

        def scale_write(i, slot):
            def row(r, _):
                for j in range(D // 16):
                    sl = (r, pl.ds(j * 16, 16))
                    rows[slot][sl] = rows[slot][sl] * SCALE
                return 0

            lax.fori_loop(0, CHUNK, row, 0)
            pltpu.sync_copy(rows[slot],
                            out_hbm.at[pl.ds(base + i * CHUNK, CHUNK)])

        fire(0, 0)

        def pair_body(g, _):
            for b in range(2):
                i = 2 * g + b

                @pl.when(i + 1 < NCHUNK)
                def _():
                    fire(i + 1, 1 - b)

                wait(b)
                scale_write(i, b)
            return 0

        lax.fori_loop(0, NCHUNK // 2, pair_body, 0)

    return gather_kernel


_GATHER = _gather_fn()


def kernel(x, table):
    idx = x.reshape(-1).astype(jnp.int32)
    emb = _GATHER(idx, table).reshape(BATCH, SEQ, D)
    return jnp.transpose(emb, (1, 0, 2))
